# baseline (speedup 1.0000x reference)
import functools

import jax
import jax.numpy as jnp
from jax import lax
from jax.experimental import pallas as pl
from jax.experimental.pallas import tpu as pltpu
from jax.experimental.pallas import tpu_sc as plsc

_NUM_CORES = 2
_NUM_SUBCORES = 16
_NUM_WORKERS = _NUM_CORES * _NUM_SUBCORES
_NBUF = 4


@functools.lru_cache(maxsize=None)
def _make_gather(batch, seq, d):
    per_worker = batch // _NUM_WORKERS
    npj = per_worker // _NBUF
    assert per_worker % _NBUF == 0 and npj >= 2
    mesh = plsc.VectorSubcoreMesh(core_axis_name="c", subcore_axis_name="s")

    @functools.partial(
        pl.kernel,
        mesh=mesh,
        compiler_params=pltpu.CompilerParams(use_tc_tiling_on_sc=False),
        out_type=jax.ShapeDtypeStruct((batch, seq, d), jnp.float32),
        scratch_types=[
            pltpu.VMEM((_NBUF, seq), jnp.int32),
            pltpu.VMEM((_NBUF, seq, d), jnp.float32),
        ] + [pltpu.SemaphoreType.DMA] * (2 * _NBUF),
    )
    def gather_kernel(table_hbm, x_hbm, out_hbm, idxb, rowsb,
                      g0, g1, g2, g3, w0, w1, w2, w3):
        gs = (g0, g1, g2, g3)
        ws = (w0, w1, w2, w3)
        wid = lax.axis_index("s") * _NUM_CORES + lax.axis_index("c")
        base = wid * per_worker

        def fire_gather(c, b):
            pltpu.sync_copy(x_hbm.at[base + c], idxb.at[b])
            pltpu.async_copy(table_hbm.at[idxb.at[b]], rowsb.at[b], gs[b])

        def wait_gather(b):
            pltpu.make_async_copy(table_hbm.at[idxb.at[b]], rowsb.at[b],
                                  gs[b]).wait()

        def fire_write(c, b):
            pltpu.async_copy(rowsb.at[b], out_hbm.at[base + c], ws[b])

        def wait_write(c, b):
            pltpu.make_async_copy(rowsb.at[b], out_hbm.at[base + c],
                                  ws[b]).wait()

        fire_gather(0, 0)
        fire_gather(1, 1)

        def slot(c, b):
            bn = (b + 2) % _NBUF
            wait_write(c - 2, bn)
            fire_gather(c + 2, bn)
            wait_gather(b)
            fire_write(c, b)

        fire_gather(2, 2)
        wait_gather(0)
        fire_write(0, 0)
        fire_gather(3, 3)
        wait_gather(1)
        fire_write(1, 1)
        slot(2, 2)
        slot(3, 3)

        def body(j, carry):
            c = j * _NBUF
            slot(c, 0)
            slot(c + 1, 1)
            slot(c + 2, 2)
            slot(c + 3, 3)
            return carry

        lax.fori_loop(1, npj - 1, body, 0)

        c = per_worker - 4
        slot(c, 0)
        slot(c + 1, 1)
        wait_gather(2)
        fire_write(c + 2, 2)
        wait_gather(3)
        fire_write(c + 3, 3)
        for b in range(_NBUF):
            wait_write(per_worker - 4 + b, b)

    return gather_kernel


@jax.jit
def kernel(x, table):
    batch, seq = x.shape
    d = table.shape[1]
    return _make_gather(batch, seq, d)(table, x.astype(jnp.int32))
